# baseline (speedup 1.0000x reference)
import functools

import jax
import jax.numpy as jnp
from jax import lax
from jax.experimental import pallas as pl
from jax.experimental.pallas import tpu as pltpu
from jax.experimental.pallas import tpu_sc as plsc

S = 4
K5 = 5
SS = S * S
KK = K5 * K5
D = SS * KK


def _argmax_body(keys_ref, q_ref, out_ref, *, n):
    sim = lax.dot_general(
        keys_ref[...], q_ref[0],
        dimension_numbers=(((1,), (0,)), ((), ())),
        preferred_element_type=jnp.float32,
    )
    m = jnp.max(sim, axis=0, keepdims=True)
    mask = (sim == m).astype(jnp.bfloat16)
    rid = lax.broadcasted_iota(jnp.int32, (8, n), 1)
    srow = lax.broadcasted_iota(jnp.int32, (8, n), 0)
    hi = (rid // 256) * 256
    lo = rid % 256
    hl = jnp.where(srow == 0, hi,
                   jnp.where(srow == 1, lo, 0)).astype(jnp.bfloat16)
    ihl = lax.dot_general(hl, mask,
                          dimension_numbers=(((1,), (0,)), ((), ())),
                          preferred_element_type=jnp.float32)
    out_ref[0, 0, :] = (ihl[0] + ihl[1]).astype(jnp.int32)


def _argmax_idx(queries, keys, p_tile=512):
    B, L, H, W = queries.shape
    n = keys.shape[0]
    hw = H * W
    nb = hw // p_tile
    qf = queries.reshape(B, L, hw).astype(jnp.bfloat16)
    keys = keys.astype(jnp.bfloat16)
    grid = (B * nb,)
    out = pl.pallas_call(
        functools.partial(_argmax_body, n=n),
        grid=grid,
        in_specs=[
            pl.BlockSpec((n, L), lambda g: (0, 0)),
            pl.BlockSpec((1, L, p_tile), lambda g, _nb=nb: (g // _nb, 0, g % _nb)),
        ],
        out_specs=pl.BlockSpec((1, 1, p_tile), lambda g: (g, 0, 0)),
        out_shape=jax.ShapeDtypeStruct((B * nb, 1, p_tile), jnp.int32),
    )(keys, qf)
    return out.reshape(B * hw)


def _make_sc_gather(n_rows, d, b_tot):
    info = plsc.get_sparse_core_info()
    nw = info.num_cores * info.num_subcores
    b_per_w = b_tot // nw
    ch = 112
    n_chunks = b_per_w // ch
    assert b_per_w % ch == 0 and b_per_w % 8 == 0

    mesh = plsc.VectorSubcoreMesh(core_axis_name="c", subcore_axis_name="s")

    @functools.partial(
        pl.kernel,
        mesh=mesh,
        out_type=jax.ShapeDtypeStruct((b_tot, d), jnp.uint32),
        scratch_types=[
            pltpu.VMEM((b_per_w,), jnp.int32),
            pltpu.VMEM((ch, d), jnp.uint32),
            pltpu.VMEM((ch, d), jnp.uint32),
            pltpu.SemaphoreType.DMA,
            pltpu.SemaphoreType.DMA,
        ],
    )
    def gather_k(table_hbm, idx_hbm, out_hbm, idx_v, rows0, rows1, s0, s1):
        wid = lax.axis_index("s") * info.num_cores + lax.axis_index("c")
        base = wid * b_per_w
        pltpu.sync_copy(idx_hbm.at[pl.ds(base, b_per_w)], idx_v)

        bufs = (rows0, rows1)
        sems = (s0, s1)
        pltpu.async_copy(table_hbm.at[idx_v.at[pl.ds(0, ch)]], rows0, s0)

        def body(i, carry):
            for par in range(2):
                @pl.when(lax.rem(i, 2) == par)
                def _():
                    nxt = 1 - par
                    @pl.when(i + 1 < n_chunks)
                    def _():
                        pltpu.async_copy(
                            table_hbm.at[idx_v.at[pl.ds((i + 1) * ch, ch)]],
                            bufs[nxt], sems[nxt])
                    pltpu.make_async_copy(
                        table_hbm.at[pl.ds(0, ch)], bufs[par], sems[par]
                    ).wait()
                    pltpu.sync_copy(bufs[par],
                                    out_hbm.at[pl.ds(base + i * ch, ch)])
            return carry

        lax.fori_loop(0, n_chunks, body, 0)

    return gather_k


def _tr_body(g_ref, out_ref):
    out_ref[...] = jnp.transpose(
        g_ref[...][:, :, :D], (2, 0, 1)).astype(jnp.float32)


def _transpose_planes(gathered, B, H, W, rh=8):
    gd = gathered.shape[-1]
    g3 = gathered.reshape(B * H, W, gd)
    blocks = (B * H) // rh
    return pl.pallas_call(
        _tr_body,
        grid=(blocks,),
        in_specs=[pl.BlockSpec((rh, W, gd), lambda g: (g, 0, 0))],
        out_specs=pl.BlockSpec((D, rh, W), lambda g: (0, g, 0)),
        out_shape=jax.ShapeDtypeStruct((D, B * H, W), jnp.float32),
    )(g3)


def _conv_body(xsh_ref, gt_ref, pc_ref, pr_ref, out_ref, *, rh, h_blocks, w):
    gidx = pl.program_id(0)
    b = gidx // h_blocks
    h0 = pl.multiple_of((gidx % h_blocks) * rh, rh)
    for c in range(3):
        xs = xsh_ref[b, c, :, pl.ds(h0, rh), :]
        accs = []
        for ss in range(SS):
            gts = gt_ref[pl.ds(ss * KK, KK)]
            accs.append(jnp.sum(xs * gts, axis=0))
        t_pre = None
        for j in range(S):
            a_j = jnp.concatenate(
                [accs[i * S + j] for i in range(S)], axis=0)
            m = lax.dot_general(
                a_j.astype(jnp.bfloat16), pc_ref[j],
                dimension_numbers=(((1,), (0,)), ((), ())),
                preferred_element_type=jnp.float32,
            )
            t_pre = m if t_pre is None else t_pre + m
        t2 = lax.dot_general(
            pr_ref[...], t_pre.astype(jnp.bfloat16),
            dimension_numbers=(((1,), (0,)), ((), ())),
            preferred_element_type=jnp.float32,
        )
        out_ref[0, c, 0] = t2


def _patch_conv(x_sh, gt, B, C, H, W, rh=8):
    h_blocks = H // rh
    rows = S * rh
    cols = S * W
    wv = jnp.arange(W)
    lv = jnp.arange(cols)
    pc = jnp.stack([(lv[None, :] == (wv[:, None] * S + j)).astype(jnp.bfloat16)
                    for j in range(S)])
    ri = jnp.arange(rows)
    pr = (((ri % S) * rh + ri // S)[:, None] ==
          jnp.arange(rows)[None, :]).astype(jnp.bfloat16)
    grid = (B * h_blocks,)
    out = pl.pallas_call(
        functools.partial(_conv_body, rh=rh, h_blocks=h_blocks, w=W),
        grid=grid,
        in_specs=[
            pl.BlockSpec(x_sh.shape, lambda g: (0, 0, 0, 0, 0)),
            pl.BlockSpec((D, rh, W), lambda g, _hb=h_blocks: (0, g, 0)),
            pl.BlockSpec(pc.shape, lambda g: (0, 0, 0)),
            pl.BlockSpec(pr.shape, lambda g: (0, 0)),
        ],
        out_specs=pl.BlockSpec(
            (1, C, 1, rows, cols),
            lambda g, _hb=h_blocks: (g // _hb, 0, g % _hb, 0, 0),
        ),
        out_shape=jax.ShapeDtypeStruct((B, C, h_blocks, rows, cols),
                                       jnp.float32),
    )(x_sh, gt, pc, pr)
    return out.reshape(B, C, H * S, W * S)


def kernel(x, queries, keys, values):
    B, C, H, W = x.shape
    n = keys.shape[0]
    pad = K5 // 2

    idx = _argmax_idx(queries, keys)

    dpad = 512
    table16 = jnp.pad(values.reshape(n, D).astype(jnp.bfloat16),
                      ((0, 0), (0, dpad - D)))
    table32 = lax.bitcast_convert_type(
        table16.reshape(n, dpad // 2, 2), jnp.uint32)
    g32 = _make_sc_gather(n, dpad // 2, B * H * W)(table32, idx)
    gathered = lax.bitcast_convert_type(
        g32, jnp.bfloat16).reshape(B * H * W, dpad)

    x_pad = jnp.pad(x, ((0, 0), (0, 0), (pad, pad), (pad, pad)),
                    mode="reflect")
    x_sh = jnp.stack([x_pad[:, :, i:i + H, j:j + W]
                      for i in range(K5) for j in range(K5)], axis=2)
    gt = _transpose_planes(gathered, B, H, W)
    return _patch_conv(x_sh, gt, B, C, H, W)


# speedup vs baseline: 2.0628x; 2.0628x over previous
import functools

import jax
import jax.numpy as jnp
from jax import lax
from jax.experimental import pallas as pl
from jax.experimental.pallas import tpu as pltpu
from jax.experimental.pallas import tpu_sc as plsc

S = 4
K5 = 5
SS = S * S
KK = K5 * K5
D = SS * KK


def _argmax_body(keys_ref, q_ref, out_ref, *, n):
    sim = lax.dot_general(
        keys_ref[...], q_ref[0],
        dimension_numbers=(((1,), (0,)), ((), ())),
        preferred_element_type=jnp.float32,
    )
    m = jnp.max(sim, axis=0, keepdims=True)
    mask = (sim == m).astype(jnp.bfloat16)
    rid = lax.broadcasted_iota(jnp.int32, (8, n), 1)
    srow = lax.broadcasted_iota(jnp.int32, (8, n), 0)
    hi = (rid // 256) * 256
    lo = rid % 256
    hl = jnp.where(srow == 0, hi,
                   jnp.where(srow == 1, lo, 0)).astype(jnp.bfloat16)
    ihl = lax.dot_general(hl, mask,
                          dimension_numbers=(((1,), (0,)), ((), ())),
                          preferred_element_type=jnp.float32)
    out_ref[0, 0, :] = (ihl[0] + ihl[1]).astype(jnp.int32)


def _argmax_idx(queries, keys, p_tile=512):
    B, L, H, W = queries.shape
    n = keys.shape[0]
    hw = H * W
    nb = hw // p_tile
    qf = queries.reshape(B, L, hw).astype(jnp.bfloat16)
    keys = keys.astype(jnp.bfloat16)
    grid = (B * nb,)
    out = pl.pallas_call(
        functools.partial(_argmax_body, n=n),
        grid=grid,
        in_specs=[
            pl.BlockSpec((n, L), lambda g: (0, 0)),
            pl.BlockSpec((1, L, p_tile), lambda g, _nb=nb: (g // _nb, 0, g % _nb)),
        ],
        out_specs=pl.BlockSpec((1, 1, p_tile), lambda g: (g, 0, 0)),
        out_shape=jax.ShapeDtypeStruct((B * nb, 1, p_tile), jnp.int32),
    )(keys, qf)
    return out.reshape(B * hw)


def _make_sc_gather(n_rows, d, b_tot):
    info = plsc.get_sparse_core_info()
    nw = info.num_cores * info.num_subcores
    b_per_w = b_tot // nw
    ch = 112
    n_chunks = b_per_w // ch
    assert b_per_w % ch == 0 and b_per_w % 8 == 0

    mesh = plsc.VectorSubcoreMesh(core_axis_name="c", subcore_axis_name="s")

    @functools.partial(
        pl.kernel,
        mesh=mesh,
        out_type=jax.ShapeDtypeStruct((b_tot, d), jnp.uint32),
        scratch_types=[
            pltpu.VMEM((b_per_w,), jnp.int32),
            pltpu.VMEM((ch, d), jnp.uint32),
            pltpu.VMEM((ch, d), jnp.uint32),
            pltpu.SemaphoreType.DMA,
            pltpu.SemaphoreType.DMA,
        ],
    )
    def gather_k(table_hbm, idx_hbm, out_hbm, idx_v, rows0, rows1, s0, s1):
        wid = lax.axis_index("s") * info.num_cores + lax.axis_index("c")
        base = wid * b_per_w
        pltpu.sync_copy(idx_hbm.at[pl.ds(base, b_per_w)], idx_v)

        bufs = (rows0, rows1)
        sems = (s0, s1)
        pltpu.async_copy(table_hbm.at[idx_v.at[pl.ds(0, ch)]], rows0, s0)

        def body(i, carry):
            for par in range(2):
                @pl.when(lax.rem(i, 2) == par)
                def _():
                    nxt = 1 - par
                    @pl.when(i + 1 < n_chunks)
                    def _():
                        pltpu.async_copy(
                            table_hbm.at[idx_v.at[pl.ds((i + 1) * ch, ch)]],
                            bufs[nxt], sems[nxt])
                    pltpu.make_async_copy(
                        table_hbm.at[pl.ds(0, ch)], bufs[par], sems[par]
                    ).wait()
                    pltpu.sync_copy(bufs[par],
                                    out_hbm.at[pl.ds(base + i * ch, ch)])
            return carry

        lax.fori_loop(0, n_chunks, body, 0)

    return gather_k


def _tr_body(g_ref, out_ref, *, rh, w):
    gv = jnp.transpose(g_ref[...], (2, 0, 1))
    lo = lax.bitcast_convert_type(gv << jnp.uint32(16), jnp.float32)
    hi = lax.bitcast_convert_type(gv & jnp.uint32(0xFFFF0000), jnp.float32)
    out_ref[pl.ds(0, 256)] = lo
    out_ref[pl.ds(256, D - 256)] = hi[:D - 256]


def _transpose_planes(g32, B, H, W, rh=8):
    gd = g32.shape[-1]
    g3 = g32.reshape(B * H, W, gd)
    blocks = (B * H) // rh
    return pl.pallas_call(
        functools.partial(_tr_body, rh=rh, w=W),
        grid=(blocks,),
        in_specs=[pl.BlockSpec((rh, W, gd), lambda g: (g, 0, 0))],
        out_specs=pl.BlockSpec((D, rh, W), lambda g: (0, g, 0)),
        out_shape=jax.ShapeDtypeStruct((D, B * H, W), jnp.float32),
    )(g3)


def _conv_body(xsh_ref, gt_ref, pc_ref, pr_ref, out_ref, *, rh, h_blocks, w):
    gidx = pl.program_id(0)
    b = gidx // h_blocks
    h0 = pl.multiple_of((gidx % h_blocks) * rh, rh)
    for c in range(3):
        xs = xsh_ref[b, c, :, pl.ds(h0, rh), :]
        accs = []
        for ss in range(SS):
            gts = gt_ref[pl.ds(ss * KK, KK)]
            accs.append(jnp.sum(xs * gts, axis=0))
        t_pre = None
        for j in range(S):
            a_j = jnp.concatenate(
                [accs[i * S + j] for i in range(S)], axis=0)
            m = lax.dot_general(
                a_j.astype(jnp.bfloat16), pc_ref[j],
                dimension_numbers=(((1,), (0,)), ((), ())),
                preferred_element_type=jnp.float32,
            )
            t_pre = m if t_pre is None else t_pre + m
        t2 = lax.dot_general(
            pr_ref[...], t_pre.astype(jnp.bfloat16),
            dimension_numbers=(((1,), (0,)), ((), ())),
            preferred_element_type=jnp.float32,
        )
        out_ref[0, c, 0] = t2


def _patch_conv(x_sh, gt, B, C, H, W, rh=8):
    h_blocks = H // rh
    rows = S * rh
    cols = S * W
    wv = jnp.arange(W)
    lv = jnp.arange(cols)
    pc = jnp.stack([(lv[None, :] == (wv[:, None] * S + j)).astype(jnp.bfloat16)
                    for j in range(S)])
    ri = jnp.arange(rows)
    pr = (((ri % S) * rh + ri // S)[:, None] ==
          jnp.arange(rows)[None, :]).astype(jnp.bfloat16)
    grid = (B * h_blocks,)
    out = pl.pallas_call(
        functools.partial(_conv_body, rh=rh, h_blocks=h_blocks, w=W),
        grid=grid,
        in_specs=[
            pl.BlockSpec(x_sh.shape, lambda g: (0, 0, 0, 0, 0)),
            pl.BlockSpec((D, rh, W), lambda g, _hb=h_blocks: (0, g, 0)),
            pl.BlockSpec(pc.shape, lambda g: (0, 0, 0)),
            pl.BlockSpec(pr.shape, lambda g: (0, 0)),
        ],
        out_specs=pl.BlockSpec(
            (1, C, 1, rows, cols),
            lambda g, _hb=h_blocks: (g // _hb, 0, g % _hb, 0, 0),
        ),
        out_shape=jax.ShapeDtypeStruct((B, C, h_blocks, rows, cols),
                                       jnp.float32),
    )(x_sh, gt, pc, pr)
    return out.reshape(B, C, H * S, W * S)


def kernel(x, queries, keys, values):
    B, C, H, W = x.shape
    n = keys.shape[0]
    pad = K5 // 2

    idx = _argmax_idx(queries, keys)

    dpad = 512
    table16 = jnp.pad(values.reshape(n, D).astype(jnp.bfloat16),
                      ((0, 0), (0, dpad - D)))
    au = lax.bitcast_convert_type(table16[:, :dpad // 2],
                                  jnp.uint16).astype(jnp.uint32)
    bu = lax.bitcast_convert_type(table16[:, dpad // 2:],
                                  jnp.uint16).astype(jnp.uint32)
    table32 = au | (bu << jnp.uint32(16))
    g32 = _make_sc_gather(n, dpad // 2, B * H * W)(table32, idx)

    x_pad = jnp.pad(x, ((0, 0), (0, 0), (pad, pad), (pad, pad)),
                    mode="reflect")
    x_sh = jnp.stack([x_pad[:, :, i:i + H, j:j + W]
                      for i in range(K5) for j in range(K5)], axis=2)
    gt = _transpose_planes(g32, B, H, W)
    return _patch_conv(x_sh, gt, B, C, H, W)
